# baseline (speedup 1.0000x reference)
import functools

import jax
import jax.numpy as jnp
from jax import lax
from jax.experimental import pallas as pl
from jax.experimental.pallas import tpu as pltpu
from jax.experimental.pallas import tpu_sc as plsc

D = 2048
I = 1024
E = 8
K = 2
T = 4096
NSH = 2

BP = 128
P0 = T * K
P = P0 + E * BP
NB = P // BP

NC = 2
NS = 16
NW = NC * NS
CT = 16


def _gate_body(x_ref, wg_ref, w_ref, i_ref):
    s = jnp.dot(x_ref[...], wg_ref[...], preferred_element_type=jnp.float32)
    m = jnp.max(s, axis=-1, keepdims=True)
    e = jnp.exp(s - m)
    p = e / jnp.sum(e, axis=-1, keepdims=True)
    ecol = lax.broadcasted_iota(jnp.int32, s.shape, 1)
    i1 = jnp.min(jnp.where(s == m, ecol, E), axis=-1, keepdims=True)
    w1 = jnp.max(p, axis=-1, keepdims=True)
    mask1 = ecol == i1
    s2 = jnp.where(mask1, -jnp.inf, s)
    m2 = jnp.max(s2, axis=-1, keepdims=True)
    i2 = jnp.min(jnp.where(s2 == m2, ecol, E), axis=-1, keepdims=True)
    w2 = jnp.max(jnp.where(mask1, -1.0, p), axis=-1, keepdims=True)
    w_ref[...] = jnp.concatenate([w1, w2], axis=-1)
    i_ref[...] = jnp.concatenate([i1, i2], axis=-1)


def _gate(xf, Wg):
    bt = 512
    return pl.pallas_call(
        _gate_body,
        grid=(T // bt,),
        in_specs=[
            pl.BlockSpec((bt, D), lambda t: (t, 0)),
            pl.BlockSpec((D, E), lambda t: (0, 0)),
        ],
        out_specs=[
            pl.BlockSpec((bt, K), lambda t: (t, 0)),
            pl.BlockSpec((bt, K), lambda t: (t, 0)),
        ],
        out_shape=[
            jax.ShapeDtypeStruct((T, K), jnp.float32),
            jax.ShapeDtypeStruct((T, K), jnp.int32),
        ],
    )(xf, Wg)


def _routing(indices):
    e_flat = indices.reshape(-1)
    order = jnp.argsort(e_flat, stable=True)
    counts = jnp.zeros((E,), jnp.int32).at[e_flat].add(1)
    starts = jnp.concatenate([jnp.zeros((1,), jnp.int32),
                              jnp.cumsum(counts)[:-1]])
    padded = ((counts + BP - 1) // BP) * BP
    ends = jnp.cumsum(padded)
    offs = jnp.concatenate([jnp.zeros((1,), jnp.int32), ends[:-1]])
    shift = offs - starts
    slot_sorted = jnp.arange(P0, dtype=jnp.int32) + shift[e_flat[order]]
    pos_flat = jnp.zeros((P0,), jnp.int32).at[order].set(slot_sorted)
    pos = pos_flat.reshape(T, K)
    block_expert = jnp.minimum(
        jnp.searchsorted(ends, jnp.arange(NB, dtype=jnp.int32) * BP,
                         side='right').astype(jnp.int32), E - 1)
    return pos[:, 0], pos[:, 1], block_expert


def _scatter_x_body(x_hbm, pos0_hbm, pos1_hbm, out_hbm,
                    rows_a, rows_b, i0a, i1a, i0b, i1b,
                    s0a, s1a, s0b, s1b):
    wid = lax.axis_index("s") * NC + lax.axis_index("c")
    tw = T // NW
    base = wid * tw
    bufs = [(rows_a, i0a, i1a, s0a, s1a), (rows_b, i0b, i1b, s0b, s1b)]
    nch = tw // CT
    pending = [None, None]
    for j in range(nch):
        rows, i0, i1, s0, s1 = bufs[j % 2]
        if pending[j % 2] is not None:
            pending[j % 2][0].wait()
            pending[j % 2][1].wait()
        t0 = base + j * CT
        pltpu.sync_copy(x_hbm.at[pl.ds(t0, CT)], rows)
        pltpu.sync_copy(pos0_hbm.at[pl.ds(t0, CT)], i0)
        pltpu.sync_copy(pos1_hbm.at[pl.ds(t0, CT)], i1)
        pending[j % 2] = (pltpu.async_copy(rows, out_hbm.at[i0], s0),
                          pltpu.async_copy(rows, out_hbm.at[i1], s1))
    for pend in pending:
        if pend is not None:
            pend[0].wait()
            pend[1].wait()


@functools.cache
def _build_scatter_x():
    return pl.kernel(
        _scatter_x_body,
        out_type=jax.ShapeDtypeStruct((P, D), jnp.float32),
        mesh=plsc.VectorSubcoreMesh(core_axis_name="c", subcore_axis_name="s"),
        scratch_types=[
            pltpu.VMEM((CT, D), jnp.float32),
            pltpu.VMEM((CT, D), jnp.float32),
            pltpu.VMEM((CT,), jnp.int32),
            pltpu.VMEM((CT,), jnp.int32),
            pltpu.VMEM((CT,), jnp.int32),
            pltpu.VMEM((CT,), jnp.int32),
            pltpu.SemaphoreType.DMA,
            pltpu.SemaphoreType.DMA,
            pltpu.SemaphoreType.DMA,
            pltpu.SemaphoreType.DMA,
        ],
    )


def _scatter_x(xf, pos0, pos1):
    return _build_scatter_x()(xf, pos0, pos1)


def _gather_body(src_hbm, idx_hbm, out_hbm, rows_a, rows_b, idx_a, idx_b,
                 ga, gb, sa, sb):
    wid = lax.axis_index("s") * NC + lax.axis_index("c")
    bw = P0 // NW
    base = wid * bw
    bufs = [(rows_a, idx_a, ga, sa), (rows_b, idx_b, gb, sb)]
    nch = bw // CT
    gat = [None] * nch
    sto = [None] * nch
    for j in range(nch):
        rows, idx, gsem, ssem = bufs[j % 2]
        if j >= 2:
            sto[j - 2].wait()
        b0 = base + j * CT
        pltpu.sync_copy(idx_hbm.at[pl.ds(b0, CT)], idx)
        gat[j] = pltpu.async_copy(src_hbm.at[idx], rows, gsem)
        if j >= 1:
            gat[j - 1].wait()
            rows_p, _, _, ssem_p = bufs[(j - 1) % 2]
            sto[j - 1] = pltpu.async_copy(
                rows_p, out_hbm.at[pl.ds(base + (j - 1) * CT, CT)], ssem_p)
    gat[nch - 1].wait()
    rows_l, _, _, ssem_l = bufs[(nch - 1) % 2]
    sto[nch - 1] = pltpu.async_copy(
        rows_l, out_hbm.at[pl.ds(base + (nch - 1) * CT, CT)], ssem_l)
    sto[nch - 2].wait()
    sto[nch - 1].wait()


@functools.cache
def _build_gather_out():
    return pl.kernel(
        _gather_body,
        out_type=jax.ShapeDtypeStruct((P0, D), jnp.float32),
        mesh=plsc.VectorSubcoreMesh(core_axis_name="c", subcore_axis_name="s"),
        scratch_types=[
            pltpu.VMEM((CT, D), jnp.float32),
            pltpu.VMEM((CT, D), jnp.float32),
            pltpu.VMEM((CT,), jnp.int32),
            pltpu.VMEM((CT,), jnp.int32),
            pltpu.SemaphoreType.DMA,
            pltpu.SemaphoreType.DMA,
            pltpu.SemaphoreType.DMA,
            pltpu.SemaphoreType.DMA,
        ],
    )


def _gather_out(src, idx):
    return _build_gather_out()(src, idx)


def _mm_body(be_ref, x_ref, w1_ref, w3_ref, w2_ref, o_ref):
    xb = x_ref[...].astype(jnp.bfloat16)
    u = jnp.dot(xb, w1_ref[0], preferred_element_type=jnp.float32)
    g = jnp.dot(xb, w3_ref[0], preferred_element_type=jnp.float32)
    h = ((u * jax.nn.sigmoid(u)) * g).astype(jnp.bfloat16)
    o_ref[...] = jnp.dot(h, w2_ref[0], preferred_element_type=jnp.float32)


def _routed_mm(block_expert, x_sorted, W1, W3, W2):
    spec = pltpu.PrefetchScalarGridSpec(
        num_scalar_prefetch=1,
        grid=(NB,),
        in_specs=[
            pl.BlockSpec((BP, D), lambda b, be: (b, 0)),
            pl.BlockSpec((1, D, I), lambda b, be: (be[b], 0, 0)),
            pl.BlockSpec((1, D, I), lambda b, be: (be[b], 0, 0)),
            pl.BlockSpec((1, I, D), lambda b, be: (be[b], 0, 0)),
        ],
        out_specs=pl.BlockSpec((BP, D), lambda b, be: (b, 0)),
    )
    return pl.pallas_call(
        _mm_body,
        grid_spec=spec,
        out_shape=jax.ShapeDtypeStruct((P, D), jnp.float32),
    )(block_expert, x_sorted, W1, W3, W2)


def _shared_body(x_ref, w1_ref, w3_ref, w2_ref, g1_ref, g2_ref, wa_ref,
                 wb_ref, o_ref):
    xb = x_ref[...]
    u = jnp.dot(xb, w1_ref[...], preferred_element_type=jnp.float32)
    g = jnp.dot(xb, w3_ref[...], preferred_element_type=jnp.float32)
    h = (u * jax.nn.sigmoid(u)) * g
    z = jnp.dot(h, w2_ref[...], preferred_element_type=jnp.float32)
    o_ref[...] = (z + g1_ref[...] * wa_ref[...]
                  + g2_ref[...] * wb_ref[...])


def _shared_combine(xf, Ws1, Ws3, Ws2, gcat, wa, wb):
    bt = 128
    ish = NSH * I
    nt = T // bt
    return pl.pallas_call(
        _shared_body,
        grid=(nt,),
        in_specs=[
            pl.BlockSpec((bt, D), lambda t: (t, 0)),
            pl.BlockSpec((D, ish), lambda t: (0, 0)),
            pl.BlockSpec((D, ish), lambda t: (0, 0)),
            pl.BlockSpec((ish, D), lambda t: (0, 0)),
            pl.BlockSpec((bt, D), lambda t: (t, 0)),
            pl.BlockSpec((bt, D), lambda t: (t + nt, 0)),
            pl.BlockSpec((bt, 1), lambda t: (t, 0)),
            pl.BlockSpec((bt, 1), lambda t: (t, 0)),
        ],
        out_specs=pl.BlockSpec((bt, D), lambda t: (t, 0)),
        out_shape=jax.ShapeDtypeStruct((T, D), jnp.float32),
    )(xf, Ws1, Ws3, Ws2, gcat, gcat, wa, wb)


def kernel(x, Wg, W1, W2, W3, Ws1, Ws2, Ws3):
    shape = x.shape
    xf = x.reshape(-1, shape[-1])

    weights, indices = _gate(xf, Wg)
    pos0, pos1, block_expert = _routing(indices)

    bf = jnp.bfloat16
    x_sorted = _scatter_x(xf, pos0, pos1)
    out_sorted = _routed_mm(block_expert, x_sorted, W1.astype(bf),
                            W3.astype(bf), W2.astype(bf))
    gcat = _gather_out(out_sorted, jnp.concatenate([pos0, pos1]))

    y = _shared_combine(xf, Ws1, Ws3, Ws2, gcat,
                        weights[:, 0:1], weights[:, 1:2])
    return y.reshape(shape)


# speedup vs baseline: 1.1595x; 1.1595x over previous
import functools

import jax
import jax.numpy as jnp
from jax import lax
from jax.experimental import pallas as pl
from jax.experimental.pallas import tpu as pltpu
from jax.experimental.pallas import tpu_sc as plsc

D = 2048
I = 1024
E = 8
K = 2
T = 4096
NSH = 2

BP = 128
P0 = T * K
P = P0 + E * BP
NB = P // BP

NC = 2
NS = 16
NW = NC * NS
CT = 16


def _gate_body(x_ref, wg_ref, w_ref, i_ref):
    s = jnp.dot(x_ref[...], wg_ref[...], preferred_element_type=jnp.float32)
    m = jnp.max(s, axis=-1, keepdims=True)
    e = jnp.exp(s - m)
    p = e / jnp.sum(e, axis=-1, keepdims=True)
    ecol = lax.broadcasted_iota(jnp.int32, s.shape, 1)
    i1 = jnp.min(jnp.where(s == m, ecol, E), axis=-1, keepdims=True)
    w1 = jnp.max(p, axis=-1, keepdims=True)
    mask1 = ecol == i1
    s2 = jnp.where(mask1, -jnp.inf, s)
    m2 = jnp.max(s2, axis=-1, keepdims=True)
    i2 = jnp.min(jnp.where(s2 == m2, ecol, E), axis=-1, keepdims=True)
    w2 = jnp.max(jnp.where(mask1, -1.0, p), axis=-1, keepdims=True)
    w_ref[...] = jnp.concatenate([w1, w2], axis=-1)
    i_ref[...] = jnp.concatenate([i1, i2], axis=-1)


def _gate(xf, Wg):
    bt = 512
    return pl.pallas_call(
        _gate_body,
        grid=(T // bt,),
        in_specs=[
            pl.BlockSpec((bt, D), lambda t: (t, 0)),
            pl.BlockSpec((D, E), lambda t: (0, 0)),
        ],
        out_specs=[
            pl.BlockSpec((bt, K), lambda t: (t, 0)),
            pl.BlockSpec((bt, K), lambda t: (t, 0)),
        ],
        out_shape=[
            jax.ShapeDtypeStruct((T, K), jnp.float32),
            jax.ShapeDtypeStruct((T, K), jnp.int32),
        ],
    )(xf, Wg)


def _routing(indices):
    e_flat = indices.reshape(-1)
    order = jnp.argsort(e_flat, stable=True)
    counts = jnp.zeros((E,), jnp.int32).at[e_flat].add(1)
    starts = jnp.concatenate([jnp.zeros((1,), jnp.int32),
                              jnp.cumsum(counts)[:-1]])
    padded = ((counts + BP - 1) // BP) * BP
    ends = jnp.cumsum(padded)
    offs = jnp.concatenate([jnp.zeros((1,), jnp.int32), ends[:-1]])
    shift = offs - starts
    slot_sorted = jnp.arange(P0, dtype=jnp.int32) + shift[e_flat[order]]
    pos_flat = jnp.zeros((P0,), jnp.int32).at[order].set(slot_sorted)
    pos = pos_flat.reshape(T, K)
    block_expert = jnp.minimum(
        jnp.searchsorted(ends, jnp.arange(NB, dtype=jnp.int32) * BP,
                         side='right').astype(jnp.int32), E - 1)
    return pos[:, 0], pos[:, 1], block_expert


def _scatter_x_body(x_hbm, pos0_hbm, pos1_hbm, out_hbm,
                    rows_a, rows_b, i0a, i1a, i0b, i1b,
                    s0a, s1a, s0b, s1b):
    wid = lax.axis_index("s") * NC + lax.axis_index("c")
    tw = T // NW
    base = wid * tw
    bufs = [(rows_a, i0a, i1a, s0a, s1a), (rows_b, i0b, i1b, s0b, s1b)]
    nch = tw // CT
    pending = [None, None]
    for j in range(nch):
        rows, i0, i1, s0, s1 = bufs[j % 2]
        if pending[j % 2] is not None:
            pending[j % 2][0].wait()
            pending[j % 2][1].wait()
        t0 = base + j * CT
        pltpu.sync_copy(x_hbm.at[pl.ds(t0, CT)], rows)
        pltpu.sync_copy(pos0_hbm.at[pl.ds(t0, CT)], i0)
        pltpu.sync_copy(pos1_hbm.at[pl.ds(t0, CT)], i1)
        pending[j % 2] = (pltpu.async_copy(rows, out_hbm.at[i0], s0),
                          pltpu.async_copy(rows, out_hbm.at[i1], s1))
    for pend in pending:
        if pend is not None:
            pend[0].wait()
            pend[1].wait()


@functools.cache
def _build_scatter_x():
    return pl.kernel(
        _scatter_x_body,
        out_type=jax.ShapeDtypeStruct((P, D), jnp.float32),
        mesh=plsc.VectorSubcoreMesh(core_axis_name="c", subcore_axis_name="s"),
        scratch_types=[
            pltpu.VMEM((CT, D), jnp.float32),
            pltpu.VMEM((CT, D), jnp.float32),
            pltpu.VMEM((CT,), jnp.int32),
            pltpu.VMEM((CT,), jnp.int32),
            pltpu.VMEM((CT,), jnp.int32),
            pltpu.VMEM((CT,), jnp.int32),
            pltpu.SemaphoreType.DMA,
            pltpu.SemaphoreType.DMA,
            pltpu.SemaphoreType.DMA,
            pltpu.SemaphoreType.DMA,
        ],
    )


def _scatter_x(xf, pos0, pos1):
    return _build_scatter_x()(xf, pos0, pos1)


def _gather_body(src_hbm, idx_hbm, out_hbm, rows_a, rows_b, idx_a, idx_b,
                 ga, gb, sa, sb):
    wid = lax.axis_index("s") * NC + lax.axis_index("c")
    bw = P0 // NW
    base = wid * bw
    bufs = [(rows_a, idx_a, ga, sa), (rows_b, idx_b, gb, sb)]
    nch = bw // CT
    gat = [None] * nch
    sto = [None] * nch
    for j in range(nch):
        rows, idx, gsem, ssem = bufs[j % 2]
        if j >= 2:
            sto[j - 2].wait()
        b0 = base + j * CT
        pltpu.sync_copy(idx_hbm.at[pl.ds(b0, CT)], idx)
        gat[j] = pltpu.async_copy(src_hbm.at[idx], rows, gsem)
        if j >= 1:
            gat[j - 1].wait()
            rows_p, _, _, ssem_p = bufs[(j - 1) % 2]
            sto[j - 1] = pltpu.async_copy(
                rows_p, out_hbm.at[pl.ds(base + (j - 1) * CT, CT)], ssem_p)
    gat[nch - 1].wait()
    rows_l, _, _, ssem_l = bufs[(nch - 1) % 2]
    sto[nch - 1] = pltpu.async_copy(
        rows_l, out_hbm.at[pl.ds(base + (nch - 1) * CT, CT)], ssem_l)
    sto[nch - 2].wait()
    sto[nch - 1].wait()


@functools.cache
def _build_gather_out():
    return pl.kernel(
        _gather_body,
        out_type=jax.ShapeDtypeStruct((P0, D), jnp.float32),
        mesh=plsc.VectorSubcoreMesh(core_axis_name="c", subcore_axis_name="s"),
        scratch_types=[
            pltpu.VMEM((CT, D), jnp.float32),
            pltpu.VMEM((CT, D), jnp.float32),
            pltpu.VMEM((CT,), jnp.int32),
            pltpu.VMEM((CT,), jnp.int32),
            pltpu.SemaphoreType.DMA,
            pltpu.SemaphoreType.DMA,
            pltpu.SemaphoreType.DMA,
            pltpu.SemaphoreType.DMA,
        ],
    )


def _gather_out(src, idx):
    return _build_gather_out()(src, idx)


def _mm_body(be_ref, x_ref, w1_ref, w3_ref, w2_ref, o_ref):
    xb = x_ref[...]
    u = jnp.dot(xb, w1_ref[0], preferred_element_type=jnp.float32)
    g = jnp.dot(xb, w3_ref[0], preferred_element_type=jnp.float32)
    h = (u * jax.nn.sigmoid(u)) * g
    o_ref[...] = jnp.dot(h, w2_ref[0], preferred_element_type=jnp.float32)


def _routed_mm(block_expert, x_sorted, W1, W3, W2):
    spec = pltpu.PrefetchScalarGridSpec(
        num_scalar_prefetch=1,
        grid=(NB,),
        in_specs=[
            pl.BlockSpec((BP, D), lambda b, be: (b, 0)),
            pl.BlockSpec((1, D, I), lambda b, be: (be[b], 0, 0)),
            pl.BlockSpec((1, D, I), lambda b, be: (be[b], 0, 0)),
            pl.BlockSpec((1, I, D), lambda b, be: (be[b], 0, 0)),
        ],
        out_specs=pl.BlockSpec((BP, D), lambda b, be: (b, 0)),
    )
    return pl.pallas_call(
        _mm_body,
        grid_spec=spec,
        out_shape=jax.ShapeDtypeStruct((P, D), jnp.float32),
    )(block_expert, x_sorted, W1, W3, W2)


def _shared_body(x_ref, w1_ref, w3_ref, w2_ref, g1_ref, g2_ref, wa_ref,
                 wb_ref, o_ref):
    xb = x_ref[...]
    u = jnp.dot(xb, w1_ref[...], preferred_element_type=jnp.float32)
    g = jnp.dot(xb, w3_ref[...], preferred_element_type=jnp.float32)
    h = (u * jax.nn.sigmoid(u)) * g
    z = jnp.dot(h, w2_ref[...], preferred_element_type=jnp.float32)
    o_ref[...] = (z + g1_ref[...] * wa_ref[...]
                  + g2_ref[...] * wb_ref[...])


def _shared_combine(xf, Ws1, Ws3, Ws2, gcat, wa, wb):
    bt = 128
    ish = NSH * I
    nt = T // bt
    return pl.pallas_call(
        _shared_body,
        grid=(nt,),
        in_specs=[
            pl.BlockSpec((bt, D), lambda t: (t, 0)),
            pl.BlockSpec((D, ish), lambda t: (0, 0)),
            pl.BlockSpec((D, ish), lambda t: (0, 0)),
            pl.BlockSpec((ish, D), lambda t: (0, 0)),
            pl.BlockSpec((bt, D), lambda t: (t, 0)),
            pl.BlockSpec((bt, D), lambda t: (t + nt, 0)),
            pl.BlockSpec((bt, 1), lambda t: (t, 0)),
            pl.BlockSpec((bt, 1), lambda t: (t, 0)),
        ],
        out_specs=pl.BlockSpec((bt, D), lambda t: (t, 0)),
        out_shape=jax.ShapeDtypeStruct((T, D), jnp.float32),
    )(xf, Ws1, Ws3, Ws2, gcat, gcat, wa, wb)


def kernel(x, Wg, W1, W2, W3, Ws1, Ws2, Ws3):
    shape = x.shape
    xf = x.reshape(-1, shape[-1])

    weights, indices = _gate(xf, Wg)
    pos0, pos1, block_expert = _routing(indices)

    x_sorted = _scatter_x(xf, pos0, pos1)
    out_sorted = _routed_mm(block_expert, x_sorted, W1, W3, W2)
    gcat = _gather_out(out_sorted, jnp.concatenate([pos0, pos1]))

    y = _shared_combine(xf, Ws1, Ws3, Ws2, gcat,
                        weights[:, 0:1], weights[:, 1:2])
    return y.reshape(shape)
